# baseline (speedup 1.0000x reference)
import functools

import jax
import jax.numpy as jnp
import numpy as np
from jax import lax
from jax.experimental import pallas as pl
from jax.experimental.pallas import tpu as pltpu
from jax.experimental.pallas import tpu_sc as plsc

_GRID_SIZES = [0.01, 0.02, 0.04, 0.08, 0.16, 0.32, 0.64, 1.28]
_IN_DIM = 3
_HID = 32
_OUT = 128
_K = 27
_ROW = 128
_N = 16384
_NPAD = _N + 512
_BLK = 512
_CH = 512
_SUB = 128
_NCHUNK = _N // _CH
_NTASK = _K * _NCHUNK
_NWORK = 32


def _offsets():
    o = np.array([[dx, dy, dz] for dx in (-1, 0, 1) for dy in (-1, 0, 1) for dz in (-1, 0, 1)], dtype=np.int64)
    return jnp.asarray(o)


def _build_nidx(p):
    b, n, _ = p.shape
    N = b * n
    offs = _offsets()
    batch_idx = jnp.repeat(jnp.arange(b, dtype=jnp.int64), n)
    out = []
    for g in _GRID_SIZES:
        gc = jnp.floor(p / g).astype(jnp.int64)
        gc = gc - gc.min(axis=1, keepdims=True)
        spatial = gc.max(axis=1).max(axis=0) + 1
        Sx, Sy, Sz = spatial[0], spatial[1], spatial[2]
        S_cap = int(np.floor(1.0 / g)) + 1
        flat = gc.reshape(-1, 3)
        keys = ((batch_idx * Sx + flat[:, 0]) * Sy + flat[:, 1]) * Sz + flat[:, 2]
        V = b * S_cap * S_cap * S_cap
        lookup = jnp.full((V,), -1, dtype=jnp.int32).at[keys].max(jnp.arange(N, dtype=jnp.int32))
        nb = flat[None, :, :] + offs[:, None, :]
        hi = spatial.astype(jnp.int64)
        inb = jnp.all((nb >= 0) & (nb < hi[None, None, :]), axis=-1)
        nbc = jnp.clip(nb, 0, hi[None, None, :] - 1)
        nkeys = ((batch_idx[None, :] * Sx + nbc[..., 0]) * Sy + nbc[..., 1]) * Sz + nbc[..., 2]
        nidx = lookup[nkeys]
        mask = inb & (nidx >= 0)
        nidxp = jnp.where(mask, nidx, _N).astype(jnp.int32)
        out.append(nidxp.reshape(_K, _NCHUNK, _CH // _SUB, _SUB))
    return out


def _pad_cols(v):
    return jnp.concatenate(
        [v, jnp.zeros((v.shape[0], _ROW - _HID), jnp.float32)], axis=1)


def _feats_body(p_ref, w1_ref, b1_ref, wl_ref, bl_ref, o_ref):
    i = pl.program_id(0)
    rows = i * _BLK + lax.broadcasted_iota(jnp.int32, (_BLK, 1), 0)
    valid = rows < _N
    pts = jnp.dot(p_ref[...], w1_ref[...], preferred_element_type=jnp.float32) + b1_ref[...]
    for s in range(len(_GRID_SIZES)):
        f = jnp.dot(pts, wl_ref[s], preferred_element_type=jnp.float32) + bl_ref[s]
        o_ref[s] = _pad_cols(jnp.where(valid, f, 0.0))


def _feats_call(p_pad, W1, b1, Wl, bl):
    L = len(_GRID_SIZES)
    return pl.pallas_call(
        _feats_body,
        grid=(_NPAD // _BLK,),
        in_specs=[
            pl.BlockSpec((_BLK, _IN_DIM), lambda i: (i, 0)),
            pl.BlockSpec((_IN_DIM, _HID), lambda i: (0, 0)),
            pl.BlockSpec((1, _HID), lambda i: (0, 0)),
            pl.BlockSpec((L, _HID, _HID), lambda i: (0, 0, 0)),
            pl.BlockSpec((L, 1, _HID), lambda i: (0, 0, 0)),
        ],
        out_specs=pl.BlockSpec((L, _BLK, _ROW), lambda i: (0, i, 0)),
        out_shape=jax.ShapeDtypeStruct((L, _NPAD, _ROW), jnp.float32),
    )(p_pad, W1, b1[None, :], Wl, bl[:, None, :])


def _dot27(g_ref, w_ref):
    acc = jnp.dot(g_ref[0], w_ref[0], preferred_element_type=jnp.float32)
    for kk in range(1, _K):
        acc += jnp.dot(g_ref[kk], w_ref[kk], preferred_element_type=jnp.float32)
    return acc


def _conv1_body(g_ref, w_ref, b_ref, r_ref, o_ref):
    i = pl.program_id(0)
    rows = i * _BLK + lax.broadcasted_iota(jnp.int32, (_BLK, 1), 0)
    valid = rows < _N
    v = _dot27(g_ref, w_ref) + b_ref[...] + r_ref[:, :_HID]
    o_ref[...] = _pad_cols(jnp.where(valid, v, 0.0))


def _conv1_call(G, Wf, bias, resid):
    return pl.pallas_call(
        _conv1_body,
        grid=(_NPAD // _BLK,),
        in_specs=[
            pl.BlockSpec((_K, _BLK, _ROW), lambda i: (0, i, 0)),
            pl.BlockSpec((_K, _ROW, _HID), lambda i: (0, 0, 0)),
            pl.BlockSpec((1, _HID), lambda i: (0, 0)),
            pl.BlockSpec((_BLK, _ROW), lambda i: (i, 0)),
        ],
        out_specs=pl.BlockSpec((_BLK, _ROW), lambda i: (i, 0)),
        out_shape=jax.ShapeDtypeStruct((_NPAD, _ROW), jnp.float32),
    )(G, Wf, bias[None, :], resid)


def _conv2_body(g_ref, w_ref, b_ref, r1_ref, r2_ref, o_ref):
    o_ref[...] = (_dot27(g_ref, w_ref) + b_ref[...]
                  + r1_ref[:, :_HID] + r2_ref[:, :_HID])


def _conv2_call(G, Wf, bias, f1, feats_i):
    return pl.pallas_call(
        _conv2_body,
        grid=(_N // _BLK,),
        in_specs=[
            pl.BlockSpec((_K, _BLK, _ROW), lambda i: (0, i, 0)),
            pl.BlockSpec((_K, _ROW, _HID), lambda i: (0, 0, 0)),
            pl.BlockSpec((1, _HID), lambda i: (0, 0)),
            pl.BlockSpec((_BLK, _ROW), lambda i: (i, 0)),
            pl.BlockSpec((_BLK, _ROW), lambda i: (i, 0)),
        ],
        out_specs=pl.BlockSpec((_BLK, _HID), lambda i: (i, 0)),
        out_shape=jax.ShapeDtypeStruct((_N, _HID), jnp.float32),
    )(G, Wf, bias[None, :], f1, feats_i)


def _final_call(os_, W2, b2):
    L = len(_GRID_SIZES)

    def body(*refs):
        o_refs = refs[:L]
        w_ref, b_ref, out_ref = refs[L], refs[L + 1], refs[L + 2]
        x = jnp.concatenate([r[...] for r in o_refs], axis=1)
        out_ref[...] = jnp.dot(x, w_ref[...], preferred_element_type=jnp.float32) + b_ref[...]

    return pl.pallas_call(
        body,
        grid=(_N // _BLK,),
        in_specs=[pl.BlockSpec((_BLK, _HID), lambda i: (i, 0)) for _ in range(L)]
        + [
            pl.BlockSpec((L * _HID, _OUT), lambda i: (0, 0)),
            pl.BlockSpec((1, _OUT), lambda i: (0, 0)),
        ],
        out_specs=pl.BlockSpec((_BLK, _OUT), lambda i: (i, 0)),
        out_shape=jax.ShapeDtypeStruct((_N, _OUT), jnp.float32),
    )(*os_, W2, b2[None, :])


_VMESH = plsc.VectorSubcoreMesh(core_axis_name="c", subcore_axis_name="s")


@jax.jit
def _sc_gather(feats, nidx):
    @functools.partial(
        pl.kernel,
        out_type=jax.ShapeDtypeStruct((_K, _NPAD, _ROW), jnp.float32),
        mesh=_VMESH,
        scratch_types=[
            pltpu.VMEM((_CH // _SUB, _SUB), jnp.int32),
            pltpu.VMEM((_CH, _ROW), jnp.float32),
            pltpu.SemaphoreType.DMA,
        ],
    )
    def k(feats_hbm, nidx_hbm, g_hbm, idx_v, rows_v, sem):
        wid = lax.axis_index("s") * 2 + lax.axis_index("c")

        @pl.loop(0, (_NTASK + _NWORK - 1) // _NWORK)
        def _(j):
            t = j * _NWORK + wid

            @pl.when(t < _NTASK)
            def _():
                kk = t // _NCHUNK
                c = lax.rem(t, _NCHUNK)
                pltpu.sync_copy(nidx_hbm.at[kk, c], idx_v)
                copies = []
                for j2 in range(_CH // _SUB):
                    copies.append(pltpu.async_copy(
                        feats_hbm.at[idx_v.at[j2]],
                        rows_v.at[pl.ds(j2 * _SUB, _SUB)], sem))
                for cp in copies:
                    cp.wait()
                pltpu.sync_copy(
                    rows_v,
                    g_hbm.at[kk, pl.ds(c * _CH, _CH), :])

    return k(feats, nidx)


def kernel(p, params):
    b, n, _ = p.shape
    L = len(_GRID_SIZES)
    nidxs = _build_nidx(p)

    p_pad = jnp.pad(p.reshape(_N, _IN_DIM), ((0, _NPAD - _N), (0, 0)))
    feats = _feats_call(p_pad, params['W1'], params['b1'], params['Wl'], params['bl'])

    Wc = jnp.pad(params['Wc'], ((0, 0), (0, 0), (0, _ROW - _HID), (0, 0)))
    bc = params['bc']

    os_ = []
    for i in range(L):
        feats_i = feats[i]
        G1 = _sc_gather(feats_i, nidxs[i])
        f1 = _conv1_call(G1, Wc[2 * i], bc[2 * i], feats_i)
        G2 = _sc_gather(f1, nidxs[i])
        o_i = _conv2_call(G2, Wc[2 * i + 1], bc[2 * i + 1], f1, feats_i)
        os_.append(o_i)

    out = _final_call(os_, params['W2'], params['b2'])
    return out.reshape(b, n, _OUT)


# speedup vs baseline: 2.3780x; 2.3780x over previous
import functools

import jax
import jax.numpy as jnp
import numpy as np
from jax import lax
from jax.experimental import pallas as pl
from jax.experimental.pallas import tpu as pltpu
from jax.experimental.pallas import tpu_sc as plsc

_GRID_SIZES = [0.01, 0.02, 0.04, 0.08, 0.16, 0.32, 0.64, 1.28]
_IN_DIM = 3
_HID = 32
_OUT = 128
_K = 27
_ROW = 128
_N = 16384
_NPAD = _N + 512
_BLK = 512
_CH = 512
_SUB = 128
_NCHUNK = _N // _CH
_NTASK = _K * _NCHUNK
_NWORK = 32


def _offsets():
    o = np.array([[dx, dy, dz] for dx in (-1, 0, 1) for dy in (-1, 0, 1) for dz in (-1, 0, 1)], dtype=np.int64)
    return jnp.asarray(o)


def _build_nidx(p):
    b, n, _ = p.shape
    N = b * n
    offs = _offsets()
    batch_idx = jnp.repeat(jnp.arange(b, dtype=jnp.int64), n)
    out = []
    for g in _GRID_SIZES:
        gc = jnp.floor(p / g).astype(jnp.int64)
        gc = gc - gc.min(axis=1, keepdims=True)
        spatial = gc.max(axis=1).max(axis=0) + 1
        Sx, Sy, Sz = spatial[0], spatial[1], spatial[2]
        S_cap = int(np.floor(1.0 / g)) + 1
        flat = gc.reshape(-1, 3)
        keys = ((batch_idx * Sx + flat[:, 0]) * Sy + flat[:, 1]) * Sz + flat[:, 2]
        V = b * S_cap * S_cap * S_cap
        lookup = jnp.full((V,), -1, dtype=jnp.int32).at[keys].max(jnp.arange(N, dtype=jnp.int32))
        nb = flat[None, :, :] + offs[:, None, :]
        hi = spatial.astype(jnp.int64)
        inb = jnp.all((nb >= 0) & (nb < hi[None, None, :]), axis=-1)
        nbc = jnp.clip(nb, 0, hi[None, None, :] - 1)
        nkeys = ((batch_idx[None, :] * Sx + nbc[..., 0]) * Sy + nbc[..., 1]) * Sz + nbc[..., 2]
        nidx = lookup[nkeys]
        mask = inb & (nidx >= 0)
        nidxp = jnp.where(mask, nidx, _N).astype(jnp.int32)
        nidxp = jnp.pad(nidxp, ((0, 0), (0, _NPAD - _N)), constant_values=_N)
        out.append(nidxp.reshape(1, _K * _NPAD))
    return out


def _pad_cols(v):
    return jnp.concatenate(
        [v, jnp.zeros((v.shape[0], _ROW - _HID), jnp.float32)], axis=1)


def _feats_body(p_ref, w1_ref, b1_ref, wl_ref, bl_ref, o_ref):
    i = pl.program_id(0)
    rows = i * _BLK + lax.broadcasted_iota(jnp.int32, (_BLK, 1), 0)
    valid = rows < _N
    pts = jnp.dot(p_ref[...], w1_ref[...], preferred_element_type=jnp.float32) + b1_ref[...]
    for s in range(len(_GRID_SIZES)):
        f = jnp.dot(pts, wl_ref[s], preferred_element_type=jnp.float32) + bl_ref[s]
        o_ref[s] = _pad_cols(jnp.where(valid, f, 0.0))


def _feats_call(p_pad, W1, b1, Wl, bl):
    L = len(_GRID_SIZES)
    return pl.pallas_call(
        _feats_body,
        grid=(_NPAD // _BLK,),
        in_specs=[
            pl.BlockSpec((_BLK, _IN_DIM), lambda i: (i, 0)),
            pl.BlockSpec((_IN_DIM, _HID), lambda i: (0, 0)),
            pl.BlockSpec((1, _HID), lambda i: (0, 0)),
            pl.BlockSpec((L, _HID, _HID), lambda i: (0, 0, 0)),
            pl.BlockSpec((L, 1, _HID), lambda i: (0, 0, 0)),
        ],
        out_specs=pl.BlockSpec((L, _BLK, _ROW), lambda i: (0, i, 0)),
        out_shape=jax.ShapeDtypeStruct((L, _NPAD, _ROW), jnp.float32),
    )(p_pad, W1, b1[None, :], Wl, bl[:, None, :])


def _dot27(g_ref, w_ref):
    acc = jnp.dot(g_ref[0], w_ref[0], preferred_element_type=jnp.float32)
    for kk in range(1, _K):
        acc += jnp.dot(g_ref[kk], w_ref[kk], preferred_element_type=jnp.float32)
    return acc


def _conv1_body(g_ref, w_ref, b_ref, r_ref, o_ref):
    i = pl.program_id(0)
    rows = i * _BLK + lax.broadcasted_iota(jnp.int32, (_BLK, 1), 0)
    valid = rows < _N
    v = _dot27(g_ref, w_ref) + b_ref[...] + r_ref[:, :_HID]
    o_ref[...] = _pad_cols(jnp.where(valid, v, 0.0))


def _conv1_call(G, Wf, bias, resid):
    return pl.pallas_call(
        _conv1_body,
        grid=(_NPAD // _BLK,),
        in_specs=[
            pl.BlockSpec((_K, _BLK, _ROW), lambda i: (0, i, 0)),
            pl.BlockSpec((_K, _ROW, _HID), lambda i: (0, 0, 0)),
            pl.BlockSpec((1, _HID), lambda i: (0, 0)),
            pl.BlockSpec((_BLK, _ROW), lambda i: (i, 0)),
        ],
        out_specs=pl.BlockSpec((_BLK, _ROW), lambda i: (i, 0)),
        out_shape=jax.ShapeDtypeStruct((_NPAD, _ROW), jnp.float32),
    )(G, Wf, bias[None, :], resid)


def _conv2_body(g_ref, w_ref, b_ref, r1_ref, r2_ref, o_ref):
    o_ref[...] = (_dot27(g_ref, w_ref) + b_ref[...]
                  + r1_ref[:, :_HID] + r2_ref[:, :_HID])


def _conv2_call(G, Wf, bias, f1, feats_i):
    return pl.pallas_call(
        _conv2_body,
        grid=(_N // _BLK,),
        in_specs=[
            pl.BlockSpec((_K, _BLK, _ROW), lambda i: (0, i, 0)),
            pl.BlockSpec((_K, _ROW, _HID), lambda i: (0, 0, 0)),
            pl.BlockSpec((1, _HID), lambda i: (0, 0)),
            pl.BlockSpec((_BLK, _ROW), lambda i: (i, 0)),
            pl.BlockSpec((_BLK, _ROW), lambda i: (i, 0)),
        ],
        out_specs=pl.BlockSpec((_BLK, _HID), lambda i: (i, 0)),
        out_shape=jax.ShapeDtypeStruct((_N, _HID), jnp.float32),
    )(G, Wf, bias[None, :], f1, feats_i)


def _final_call(os_, W2, b2):
    L = len(_GRID_SIZES)

    def body(*refs):
        o_refs = refs[:L]
        w_ref, b_ref, out_ref = refs[L], refs[L + 1], refs[L + 2]
        x = jnp.concatenate([r[...] for r in o_refs], axis=1)
        out_ref[...] = jnp.dot(x, w_ref[...], preferred_element_type=jnp.float32) + b_ref[...]

    return pl.pallas_call(
        body,
        grid=(_N // _BLK,),
        in_specs=[pl.BlockSpec((_BLK, _HID), lambda i: (i, 0)) for _ in range(L)]
        + [
            pl.BlockSpec((L * _HID, _OUT), lambda i: (0, 0)),
            pl.BlockSpec((1, _OUT), lambda i: (0, 0)),
        ],
        out_specs=pl.BlockSpec((_BLK, _OUT), lambda i: (i, 0)),
        out_shape=jax.ShapeDtypeStruct((_N, _OUT), jnp.float32),
    )(*os_, W2, b2[None, :])


_VMESH = plsc.VectorSubcoreMesh(core_axis_name="c", subcore_axis_name="s")


@jax.jit
def _sc_gather(feats, nidx_flat):
    B = nidx_flat.shape[1]

    @functools.partial(
        pl.kernel,
        out_type=jax.ShapeDtypeStruct((B, _ROW), jnp.float32),
        mesh=_VMESH,
    )
    def k(x_hbm, i_hbm, o_hbm):
        def body(i_vmem, o_vmem):
            pltpu.sync_copy(x_hbm.at[i_vmem.at[0]], o_vmem)

        pltpu.emit_pipeline(
            body,
            grid=(B // _SUB,),
            in_specs=[pl.BlockSpec((1, _SUB), lambda i: (0, i))],
            out_specs=[pl.BlockSpec((_SUB, _ROW), lambda i: (i, 0))],
            core_axis_name=("c", "s"),
            dimension_semantics=(pltpu.PARALLEL,),
        )(i_hbm, o_hbm)

    return k(feats, nidx_flat).reshape(_K, _NPAD, _ROW)


def kernel(p, params):
    b, n, _ = p.shape
    L = len(_GRID_SIZES)
    nidxs = _build_nidx(p)

    p_pad = jnp.pad(p.reshape(_N, _IN_DIM), ((0, _NPAD - _N), (0, 0)))
    feats = _feats_call(p_pad, params['W1'], params['b1'], params['Wl'], params['bl'])

    Wc = jnp.pad(params['Wc'], ((0, 0), (0, 0), (0, _ROW - _HID), (0, 0)))
    bc = params['bc']

    os_ = []
    for i in range(L):
        feats_i = feats[i]
        G1 = _sc_gather(feats_i, nidxs[i])
        f1 = _conv1_call(G1, Wc[2 * i], bc[2 * i], feats_i)
        G2 = _sc_gather(f1, nidxs[i])
        o_i = _conv2_call(G2, Wc[2 * i + 1], bc[2 * i + 1], f1, feats_i)
        os_.append(o_i)

    out = _final_call(os_, params['W2'], params['b2'])
    return out.reshape(b, n, _OUT)
